# Initial kernel scaffold; baseline (speedup 1.0000x reference)
import functools

import jax
import jax.numpy as jnp
from jax.experimental import pallas as pl

_L = 2048
_C = 768
_K = 7
_LB = 128


def _silu(v):
    return v * jax.nn.sigmoid(v)


def _block_kernel(xpad_ref, vwin_ref, dw_ref, pwT_ref, pwb_ref,
                  woff_ref, boff_ref, wmask_ref, bmask_ref,
                  kwt_ref, woutT_ref, bout_ref,
                  out_ref, reg_ref, ent_ref):
    i = pl.program_id(0)
    l0 = i * _LB

    xl = xpad_ref[pl.ds(l0, _LB), :]
    xc = xpad_ref[pl.ds(l0 + 1, _LB), :]
    xr = xpad_ref[pl.ds(l0 + 2, _LB), :]
    h = (xl * dw_ref[0:1, :] + xc * dw_ref[1:2, :] + xr * dw_ref[2:3, :]
         + dw_ref[3:4, :])
    h = _silu(h)
    xdw = jnp.dot(h, pwT_ref[...], preferred_element_type=jnp.float32) \
        + pwb_ref[0:1, :]

    mm_off = jnp.dot(xdw, woff_ref[...], preferred_element_type=jnp.float32) \
        + boff_ref[0:1, :]
    mm_mask = jnp.dot(xdw, wmask_ref[...], preferred_element_type=jnp.float32) \
        + bmask_ref[0:1, :]

    m = [mm_mask[:, k * _C:(k + 1) * _C] for k in range(_K)]
    mmax = m[0]
    for k in range(1, _K):
        mmax = jnp.maximum(mmax, m[k])
    e = [jnp.exp(m[k] - mmax) for k in range(_K)]
    s = e[0]
    for k in range(1, _K):
        s = s + e[k]
    inv_s = 1.0 / s

    lcol = jax.lax.broadcasted_iota(jnp.float32, (_LB, 1), 0) \
        + l0.astype(jnp.float32)

    num = jnp.zeros((_LB, _C), jnp.float32)
    reg_blk = jnp.zeros((), jnp.float32)
    ent_blk = jnp.zeros((), jnp.float32)
    for k in range(_K):
        off = jnp.tanh(mm_off[:, k * _C:(k + 1) * _C]) * 2.0
        reg_blk = reg_blk + jnp.sum(off * off)
        pos = jnp.clip(lcol + jnp.float32(k - 3) + off, 0.0,
                       jnp.float32(_L - 1))
        relp = pos - lcol
        acc = jnp.zeros((_LB, _C), jnp.float32)
        for j in range(k, k + 6):
            w = jnp.maximum(1.0 - jnp.abs(relp - jnp.float32(j - 5)), 0.0)
            vc = jnp.broadcast_to(vwin_ref[:, j:j + 1], (_LB, _C))
            acc = acc + vc * w
        a = e[k] * inv_s
        ent_blk = ent_blk + jnp.sum(a * jnp.log(a + 1e-8))
        num = num + (e[k] * acc) * kwt_ref[k:k + 1, :]

    out_pre = num * inv_s
    out_ref[...] = jnp.dot(out_pre, woutT_ref[...],
                           preferred_element_type=jnp.float32) + bout_ref[0:1, :]

    reg_blk = (reg_blk / jnp.float32(_L * _C * _K)).reshape(1, 1)
    ent_blk = (ent_blk / jnp.float32(_L * _C)).reshape(1, 1)

    @pl.when(i == 0)
    def _():
        reg_ref[...] = reg_blk
        ent_ref[...] = ent_blk

    @pl.when(i != 0)
    def _():
        reg_ref[...] = reg_ref[...] + reg_blk
        ent_ref[...] = ent_ref[...] + ent_blk


@functools.partial(jax.jit, static_argnames=("interpret",))
def _run(x, params, interpret=False):
    p = params
    f32 = jnp.float32

    sigma = jnp.clip(jax.nn.softplus(p["raw_sigma"]), 0.05, 0.5)
    grid = jnp.linspace(-0.5, 0.5, _K).reshape(_K, 1)
    dist_sq = (grid / jnp.clip(sigma.reshape(1, 1), 1e-6, None)) ** 2
    env = jnp.exp(-0.5 * dist_sq.sum(-1))
    env = env / jnp.clip(env.sum(), 1e-8, None)

    kh = grid * 30.0
    kh = _silu(kh @ p["kn_W1"].T + p["kn_b1"])
    kh = _silu(kh @ p["kn_W2"].T + p["kn_b2"])
    kh = _silu(kh @ p["kn_W3"].T + p["kn_b3"])
    kernel_weights = kh @ p["kn_W4"].T + p["kn_b4"]
    idx = jnp.arange(_C)[None, :] * _K + jnp.arange(_K)[:, None]
    kwt = kernel_weights.reshape(-1)[idx]
    kwt8 = jnp.concatenate([kwt, jnp.zeros((1, _C), f32)], axis=0)

    woffP = p["W_off"].reshape(_C, _K, _C).transpose(1, 0, 2) \
        .reshape(_K * _C, _C).T
    boffP = p["b_off"].reshape(_C, _K).T.reshape(1, _K * _C)
    wmaskP = (p["W_mask"].reshape(_C, _K, _C).transpose(1, 0, 2)
              * env[:, None, None]).reshape(_K * _C, _C).T
    bmaskP = (p["b_mask"].reshape(_C, _K).T * env[:, None]).reshape(1, _K * _C)

    dwstack = jnp.concatenate(
        [p["dw_w"][:, 0, :].T, p["dw_b"].reshape(1, _C),
         jnp.zeros((4, _C), f32)], axis=0)
    pwT = p["pw_w"][:, :, 0].T
    pwb = p["pw_b"].reshape(1, _C)
    woutT = p["W_out"].T
    bout = p["b_out"].reshape(1, _C)

    v = (x[0, 0:3, :] @ p["W_in"].T + p["b_in"]).reshape(-1)[:_L]
    vp = jnp.pad(v, (5, 6), mode="edge")
    vwin = jnp.stack([vp[j:j + _L] for j in range(12)], axis=1)

    xpad = jnp.concatenate(
        [jnp.zeros((1, _C), f32), x[0], jnp.zeros((7, _C), f32)], axis=0)

    nblk = _L // _LB
    full = lambda shape: pl.BlockSpec(shape, lambda i: (0, 0))
    out, reg, ent = pl.pallas_call(
        _block_kernel,
        grid=(nblk,),
        in_specs=[
            full((_L + 8, _C)),
            pl.BlockSpec((_LB, 12), lambda i: (i, 0)),
            full((8, _C)),
            full((_C, _C)),
            full((1, _C)),
            full((_C, _K * _C)),
            full((1, _K * _C)),
            full((_C, _K * _C)),
            full((1, _K * _C)),
            full((8, _C)),
            full((_C, _C)),
            full((1, _C)),
        ],
        out_specs=[
            pl.BlockSpec((_LB, _C), lambda i: (i, 0)),
            pl.BlockSpec((1, 1), lambda i: (0, 0)),
            pl.BlockSpec((1, 1), lambda i: (0, 0)),
        ],
        out_shape=[
            jax.ShapeDtypeStruct((_L, _C), f32),
            jax.ShapeDtypeStruct((1, 1), f32),
            jax.ShapeDtypeStruct((1, 1), f32),
        ],
        interpret=interpret,
    )(xpad, vwin, dwstack, pwT, pwb, woffP, boffP, wmaskP, bmaskP,
      kwt8, woutT, bout)

    return out.reshape(1, _L, _C), reg[0, 0], ent[0, 0]


def kernel(x, params):
    return _run(x, params)


# speedup vs baseline: 3802.6703x; 3802.6703x over previous
import functools

import jax
import jax.numpy as jnp
from jax.experimental import pallas as pl

_L = 2048
_C = 768
_K = 7
_LB = 128


def _silu(v):
    return v * jax.nn.sigmoid(v)


def _block_kernel(xm_ref, xc_ref, xp_ref, vwin_ref, dw_ref, pwT_ref, pwb_ref,
                  woff_ref, boff_ref, wmask_ref, bmask_ref,
                  kwt_ref, woutT_ref, bout_ref,
                  out_ref, reg_ref, ent_ref):
    i = pl.program_id(0)
    l0 = i * _LB

    h = (xm_ref[...] * dw_ref[0:1, :] + xc_ref[...] * dw_ref[1:2, :]
         + xp_ref[...] * dw_ref[2:3, :] + dw_ref[3:4, :])
    h = _silu(h)
    xdw = jnp.dot(h, pwT_ref[...], preferred_element_type=jnp.float32) \
        + pwb_ref[0:1, :]

    mm_off = jnp.dot(xdw, woff_ref[...], preferred_element_type=jnp.float32) \
        + boff_ref[0:1, :]
    mm_mask = jnp.dot(xdw, wmask_ref[...], preferred_element_type=jnp.float32) \
        + bmask_ref[0:1, :]

    m = [mm_mask[:, k * _C:(k + 1) * _C] for k in range(_K)]
    mmax = m[0]
    for k in range(1, _K):
        mmax = jnp.maximum(mmax, m[k])
    e = [jnp.exp(m[k] - mmax) for k in range(_K)]
    s = e[0]
    for k in range(1, _K):
        s = s + e[k]
    inv_s = 1.0 / s

    lcol = (jax.lax.broadcasted_iota(jnp.int32, (_LB, 1), 0)
            + l0).astype(jnp.float32)

    num = jnp.zeros((_LB, _C), jnp.float32)
    reg_blk = jnp.zeros((), jnp.float32)
    ent_blk = jnp.zeros((), jnp.float32)
    for k in range(_K):
        off = jnp.tanh(mm_off[:, k * _C:(k + 1) * _C]) * 2.0
        reg_blk = reg_blk + jnp.sum(off * off)
        pos = jnp.clip(lcol + jnp.float32(k - 3) + off, 0.0,
                       jnp.float32(_L - 1))
        relp = pos - lcol
        acc = jnp.zeros((_LB, _C), jnp.float32)
        taps = range(5, 12) if k == _K - 1 else range(k, k + 6)
        for j in taps:
            w = jnp.maximum(1.0 - jnp.abs(relp - jnp.float32(j - 5)), 0.0)
            vc = jnp.broadcast_to(vwin_ref[:, j:j + 1], (_LB, _C))
            acc = acc + vc * w
        a = e[k] * inv_s
        ent_blk = ent_blk + jnp.sum(a * jnp.log(a + 1e-8))
        num = num + (e[k] * acc) * kwt_ref[k:k + 1, :]

    out_pre = num * inv_s
    out_ref[...] = jnp.dot(out_pre, woutT_ref[...],
                           preferred_element_type=jnp.float32) + bout_ref[0:1, :]

    reg_blk = (reg_blk / jnp.float32(_L * _C * _K)).reshape(1, 1)
    ent_blk = (ent_blk / jnp.float32(_L * _C)).reshape(1, 1)

    @pl.when(i == 0)
    def _():
        reg_ref[...] = reg_blk
        ent_ref[...] = ent_blk

    @pl.when(i != 0)
    def _():
        reg_ref[...] = reg_ref[...] + reg_blk
        ent_ref[...] = ent_ref[...] + ent_blk


@functools.partial(jax.jit, static_argnames=("interpret",))
def _run(x, params, interpret=False):
    p = params
    f32 = jnp.float32

    sigma = jnp.clip(jax.nn.softplus(p["raw_sigma"]), 0.05, 0.5)
    grid = jnp.linspace(-0.5, 0.5, _K).reshape(_K, 1)
    dist_sq = (grid / jnp.clip(sigma.reshape(1, 1), 1e-6, None)) ** 2
    env = jnp.exp(-0.5 * dist_sq.sum(-1))
    env = env / jnp.clip(env.sum(), 1e-8, None)

    kh = grid * 30.0
    kh = _silu(kh @ p["kn_W1"].T + p["kn_b1"])
    kh = _silu(kh @ p["kn_W2"].T + p["kn_b2"])
    kh = _silu(kh @ p["kn_W3"].T + p["kn_b3"])
    kernel_weights = kh @ p["kn_W4"].T + p["kn_b4"]
    idx = jnp.arange(_C)[None, :] * _K + jnp.arange(_K)[:, None]
    kwt = kernel_weights.reshape(-1)[idx]
    kwt8 = jnp.concatenate([kwt, jnp.zeros((1, _C), f32)], axis=0)

    woffP = p["W_off"].reshape(_C, _K, _C).transpose(1, 0, 2) \
        .reshape(_K * _C, _C).T
    boffP = p["b_off"].reshape(_C, _K).T.reshape(1, _K * _C)
    wmaskP = (p["W_mask"].reshape(_C, _K, _C).transpose(1, 0, 2)
              * env[:, None, None]).reshape(_K * _C, _C).T
    bmaskP = (p["b_mask"].reshape(_C, _K).T * env[:, None]).reshape(1, _K * _C)

    dwstack = jnp.concatenate(
        [p["dw_w"][:, 0, :].T, p["dw_b"].reshape(1, _C),
         jnp.zeros((4, _C), f32)], axis=0)
    pwT = p["pw_w"][:, :, 0].T
    pwb = p["pw_b"].reshape(1, _C)
    woutT = p["W_out"].T
    bout = p["b_out"].reshape(1, _C)

    v = (x[0, 0:3, :] @ p["W_in"].T + p["b_in"]).reshape(-1)[:_L]
    vp = jnp.pad(v, (5, 6), mode="edge")
    vwin = jnp.stack([vp[j:j + _L] for j in range(12)], axis=1)

    z1 = jnp.zeros((1, _C), f32)
    xm = jnp.concatenate([z1, x[0, :-1]], axis=0)
    xc = x[0]
    xp = jnp.concatenate([x[0, 1:], z1], axis=0)

    nblk = _L // _LB
    full = lambda shape: pl.BlockSpec(shape, lambda i: (0, 0))
    row_spec = pl.BlockSpec((_LB, _C), lambda i: (i, 0))
    out, reg, ent = pl.pallas_call(
        _block_kernel,
        grid=(nblk,),
        in_specs=[
            row_spec,
            row_spec,
            row_spec,
            pl.BlockSpec((_LB, 12), lambda i: (i, 0)),
            full((8, _C)),
            full((_C, _C)),
            full((1, _C)),
            full((_C, _K * _C)),
            full((1, _K * _C)),
            full((_C, _K * _C)),
            full((1, _K * _C)),
            full((8, _C)),
            full((_C, _C)),
            full((1, _C)),
        ],
        out_specs=[
            pl.BlockSpec((_LB, _C), lambda i: (i, 0)),
            pl.BlockSpec((1, 1), lambda i: (0, 0)),
            pl.BlockSpec((1, 1), lambda i: (0, 0)),
        ],
        out_shape=[
            jax.ShapeDtypeStruct((_L, _C), f32),
            jax.ShapeDtypeStruct((1, 1), f32),
            jax.ShapeDtypeStruct((1, 1), f32),
        ],
        interpret=interpret,
    )(xm, xc, xp, vwin, dwstack, pwT, pwb, woffP, boffP, wmaskP, bmaskP,
      kwt8, woutT, bout)

    return out.reshape(1, _L, _C), reg[0, 0], ent[0, 0]


def kernel(x, params):
    return _run(x, params)


# speedup vs baseline: 3868.6125x; 1.0173x over previous
import functools

import jax
import jax.numpy as jnp
from jax.experimental import pallas as pl

_L = 2048
_C = 768
_K = 7
_LB = 128


def _silu(v):
    return v * jax.nn.sigmoid(v)


def _block_kernel(xm_ref, xc_ref, xp_ref, vwin_ref, dw_ref, pwT_ref, pwb_ref,
                  woff_ref, boff_ref, wmask_ref, bmask_ref,
                  kwt_ref, woutT_ref, bout_ref,
                  out_ref, reg_ref, ent_ref):
    i = pl.program_id(0)
    l0 = i * _LB

    h = (xm_ref[...] * dw_ref[0:1, :] + xc_ref[...] * dw_ref[1:2, :]
         + xp_ref[...] * dw_ref[2:3, :] + dw_ref[3:4, :])
    h = _silu(h)
    xdw = jnp.dot(h, pwT_ref[...], preferred_element_type=jnp.float32) \
        + pwb_ref[0:1, :]

    mm_off = jnp.dot(xdw, woff_ref[...], preferred_element_type=jnp.float32) \
        + boff_ref[0:1, :]
    mm_mask = jnp.dot(xdw, wmask_ref[...], preferred_element_type=jnp.float32) \
        + bmask_ref[0:1, :]

    m = [mm_mask[:, k * _C:(k + 1) * _C] for k in range(_K)]
    mmax = m[0]
    for k in range(1, _K):
        mmax = jnp.maximum(mmax, m[k])
    e = [jnp.exp(m[k] - mmax) for k in range(_K)]
    s = e[0]
    for k in range(1, _K):
        s = s + e[k]
    inv_s = 1.0 / s

    lcol = (jax.lax.broadcasted_iota(jnp.int32, (_LB, 1), 0)
            + l0).astype(jnp.float32)

    num = jnp.zeros((_LB, _C), jnp.float32)
    reg_blk = jnp.zeros((), jnp.float32)
    ent_blk = jnp.zeros((), jnp.float32)
    for k in range(_K):
        off = jnp.tanh(mm_off[:, k * _C:(k + 1) * _C]) * 2.0
        reg_blk = reg_blk + jnp.sum(off * off)
        pos = jnp.clip(lcol + jnp.float32(k - 3) + off, 0.0,
                       jnp.float32(_L - 1))
        relp = pos - lcol
        acc = jnp.zeros((_LB, _C), jnp.float32)
        taps = range(5, 12) if k == _K - 1 else range(k, k + 6)
        for j in taps:
            w = jnp.maximum(1.0 - jnp.abs(relp - jnp.float32(j - 5)), 0.0)
            vc = jnp.broadcast_to(vwin_ref[:, j:j + 1], (_LB, _C))
            acc = acc + vc * w
        a = e[k] * inv_s
        ent_blk = ent_blk + jnp.sum(a * jnp.log(a + 1e-8))
        num = num + (e[k] * acc) * kwt_ref[k:k + 1, :]

    out_pre = num * inv_s
    out_ref[...] = jnp.dot(out_pre, woutT_ref[...],
                           preferred_element_type=jnp.float32) + bout_ref[0:1, :]

    reg_blk = (reg_blk / jnp.float32(_L * _C * _K)).reshape(1, 1)
    ent_blk = (ent_blk / jnp.float32(_L * _C)).reshape(1, 1)

    @pl.when(i == 0)
    def _():
        reg_ref[...] = reg_blk
        ent_ref[...] = ent_blk

    @pl.when(i != 0)
    def _():
        reg_ref[...] = reg_ref[...] + reg_blk
        ent_ref[...] = ent_ref[...] + ent_blk


@functools.partial(jax.jit, static_argnames=("interpret",))
def _run(x, params, interpret=False):
    p = params
    f32 = jnp.float32

    sigma = jnp.clip(jax.nn.softplus(p["raw_sigma"]), 0.05, 0.5)
    grid = jnp.linspace(-0.5, 0.5, _K).reshape(_K, 1)
    dist_sq = (grid / jnp.clip(sigma.reshape(1, 1), 1e-6, None)) ** 2
    env = jnp.exp(-0.5 * dist_sq.sum(-1))
    env = env / jnp.clip(env.sum(), 1e-8, None)

    kh = grid * 30.0
    kh = _silu(kh @ p["kn_W1"].T + p["kn_b1"])
    kh = _silu(kh @ p["kn_W2"].T + p["kn_b2"])
    kh = _silu(kh @ p["kn_W3"].T + p["kn_b3"])
    kernel_weights = kh @ p["kn_W4"].T + p["kn_b4"]
    kwt = kernel_weights.reshape(_C, _K).T
    kwt8 = jnp.concatenate([kwt, jnp.zeros((1, _C), f32)], axis=0)

    woffP = p["W_off"].reshape(_C, _K, _C).transpose(1, 0, 2) \
        .reshape(_K * _C, _C).T
    boffP = p["b_off"].reshape(_C, _K).T.reshape(1, _K * _C)
    wmaskP = (p["W_mask"].reshape(_C, _K, _C).transpose(1, 0, 2)
              * env[:, None, None]).reshape(_K * _C, _C).T
    bmaskP = (p["b_mask"].reshape(_C, _K).T * env[:, None]).reshape(1, _K * _C)

    dwstack = jnp.concatenate(
        [p["dw_w"][:, 0, :].T, p["dw_b"].reshape(1, _C),
         jnp.zeros((4, _C), f32)], axis=0)
    pwT = p["pw_w"][:, :, 0].T
    pwb = p["pw_b"].reshape(1, _C)
    woutT = p["W_out"].T
    bout = p["b_out"].reshape(1, _C)

    v = (x[0, 0:3, :] @ p["W_in"].T + p["b_in"]).reshape(-1)[:_L]
    vp = jnp.pad(v, (5, 6), mode="edge")
    vwin = jnp.stack([vp[j:j + _L] for j in range(12)], axis=1)

    z1 = jnp.zeros((1, _C), f32)
    xm = jnp.concatenate([z1, x[0, :-1]], axis=0)
    xc = x[0]
    xp = jnp.concatenate([x[0, 1:], z1], axis=0)

    nblk = _L // _LB
    full = lambda shape: pl.BlockSpec(shape, lambda i: (0, 0))
    row_spec = pl.BlockSpec((_LB, _C), lambda i: (i, 0))
    out, reg, ent = pl.pallas_call(
        _block_kernel,
        grid=(nblk,),
        in_specs=[
            row_spec,
            row_spec,
            row_spec,
            pl.BlockSpec((_LB, 12), lambda i: (i, 0)),
            full((8, _C)),
            full((_C, _C)),
            full((1, _C)),
            full((_C, _K * _C)),
            full((1, _K * _C)),
            full((_C, _K * _C)),
            full((1, _K * _C)),
            full((8, _C)),
            full((_C, _C)),
            full((1, _C)),
        ],
        out_specs=[
            pl.BlockSpec((_LB, _C), lambda i: (i, 0)),
            pl.BlockSpec((1, 1), lambda i: (0, 0)),
            pl.BlockSpec((1, 1), lambda i: (0, 0)),
        ],
        out_shape=[
            jax.ShapeDtypeStruct((_L, _C), f32),
            jax.ShapeDtypeStruct((1, 1), f32),
            jax.ShapeDtypeStruct((1, 1), f32),
        ],
        interpret=interpret,
    )(xm, xc, xp, vwin, dwstack, pwT, pwb, woffP, boffP, wmaskP, bmaskP,
      kwt8, woutT, bout)

    return out.reshape(1, _L, _C), reg[0, 0], ent[0, 0]


def kernel(x, params):
    return _run(x, params)


# speedup vs baseline: 4320.8349x; 1.1169x over previous
import functools

import jax
import jax.numpy as jnp
from jax.experimental import pallas as pl

_L = 2048
_C = 768
_K = 7
_LB = 128


def _silu(v):
    return v * jax.nn.sigmoid(v)


def _block_kernel(xpad_ref, vwin_ref, dwT_ref, pw_ref, woff_ref, wmask_ref,
                  boffT_ref, bmsT_ref, env_ref, kwtT_ref, wout_ref, boutr_ref,
                  out_ref, reg_ref, ent_ref):
    i = pl.program_id(0)
    l0 = i * _LB
    f32 = jnp.float32

    slab = xpad_ref[pl.ds(l0, _LB + 8), :]
    h = (slab[0:_LB] * dwT_ref[0:1, :] + slab[1:_LB + 1] * dwT_ref[1:2, :]
         + slab[2:_LB + 2] * dwT_ref[2:3, :] + dwT_ref[3:4, :])
    h = _silu(h)
    xdwT = jax.lax.dot_general(pw_ref[...], h, (((1,), (1,)), ((), ())),
                               preferred_element_type=f32) \
        + boffT_ref[:, 7:8]

    lrow = (jax.lax.broadcasted_iota(jnp.int32, (1, _LB), 1)
            + l0).astype(f32)

    m = []
    accs = []
    reg_blk = jnp.zeros((), f32)
    for k in range(_K):
        wk = woff_ref[:, k * _C:(k + 1) * _C]
        mo = jax.lax.dot_general(wk, xdwT, (((1,), (0,)), ((), ())),
                                 preferred_element_type=f32)
        off = jnp.tanh(mo + boffT_ref[:, k:k + 1]) * 2.0
        reg_blk = reg_blk + jnp.sum(off * off)
        pos = jnp.clip(lrow + jnp.float32(k - 3) + off, 0.0,
                       jnp.float32(_L - 1))
        relp = pos - lrow
        acc = jnp.zeros((_C, _LB), f32)
        taps = range(5, 12) if k == _K - 1 else range(k, k + 6)
        for j in taps:
            w = jnp.maximum(1.0 - jnp.abs(relp - jnp.float32(j - 5)), 0.0)
            vc = jnp.broadcast_to(vwin_ref[j:j + 1, :], (_C, _LB))
            acc = acc + vc * w
        accs.append(acc)

        wmk = wmask_ref[:, k * _C:(k + 1) * _C]
        mmk = jax.lax.dot_general(wmk, xdwT, (((1,), (0,)), ((), ())),
                                  preferred_element_type=f32)
        envk = jnp.broadcast_to(env_ref[k:k + 1, 0:1], (_C, _LB))
        m.append(mmk * envk + bmsT_ref[:, k:k + 1])

    mmax = m[0]
    for k in range(1, _K):
        mmax = jnp.maximum(mmax, m[k])
    e = [jnp.exp(m[k] - mmax) for k in range(_K)]
    s = e[0]
    for k in range(1, _K):
        s = s + e[k]
    inv_s = 1.0 / s

    num = jnp.zeros((_C, _LB), f32)
    t1 = jnp.zeros((_C, _LB), f32)
    for k in range(_K):
        num = num + (e[k] * accs[k]) * kwtT_ref[:, k:k + 1]
        t1 = t1 + e[k] * (m[k] - mmax)
    ent_blk = jnp.sum(t1 * inv_s - jnp.log(s))

    out_preT = num * inv_s
    res = jax.lax.dot_general(out_preT, wout_ref[...],
                              (((0,), (1,)), ((), ())),
                              preferred_element_type=f32)
    out_ref[...] = res + boutr_ref[0:1, :]

    reg_blk = (reg_blk / jnp.float32(_L * _C * _K)).reshape(1, 1)
    ent_blk = (ent_blk / jnp.float32(_L * _C)).reshape(1, 1)

    @pl.when(i == 0)
    def _():
        reg_ref[...] = reg_blk
        ent_ref[...] = ent_blk

    @pl.when(i != 0)
    def _():
        reg_ref[...] = reg_ref[...] + reg_blk
        ent_ref[...] = ent_ref[...] + ent_blk


@functools.partial(jax.jit, static_argnames=("interpret",))
def _run(x, params, interpret=False):
    p = params
    f32 = jnp.float32

    sigma = jnp.clip(jax.nn.softplus(p["raw_sigma"]), 0.05, 0.5)
    grid = jnp.linspace(-0.5, 0.5, _K).reshape(_K, 1)
    dist_sq = (grid / jnp.clip(sigma.reshape(1, 1), 1e-6, None)) ** 2
    env = jnp.exp(-0.5 * dist_sq.sum(-1))
    env = env / jnp.clip(env.sum(), 1e-8, None)
    env8 = jnp.zeros((8, 128), f32).at[:_K, :].set(env[:, None])

    kh = grid * 30.0
    kh = _silu(kh @ p["kn_W1"].T + p["kn_b1"])
    kh = _silu(kh @ p["kn_W2"].T + p["kn_b2"])
    kh = _silu(kh @ p["kn_W3"].T + p["kn_b3"])
    kernel_weights = kh @ p["kn_W4"].T + p["kn_b4"]
    kwtT = jnp.zeros((_C, 8), f32).at[:, :_K].set(
        kernel_weights.reshape(_C, _K))

    woff2 = p["W_off"].reshape(_C, _K * _C)
    wmask2 = p["W_mask"].reshape(_C, _K * _C)
    boffT = jnp.zeros((_C, 8), f32).at[:, :_K].set(p["b_off"].reshape(_C, _K))
    boffT = boffT.at[:, 7].set(p["pw_b"])
    bmsT = jnp.zeros((_C, 8), f32).at[:, :_K].set(
        p["b_mask"].reshape(_C, _K) * env[None, :])

    dwT = jnp.concatenate(
        [p["dw_w"][:, 0, :].T, p["dw_b"].reshape(1, _C),
         jnp.zeros((4, _C), f32)], axis=0)
    pw = p["pw_w"][:, :, 0]
    wout = p["W_out"]
    boutr = p["b_out"].reshape(1, _C)

    v = (x[0, 0:3, :] @ p["W_in"].T + p["b_in"]).reshape(-1)[:_L]
    vp = jnp.pad(v, (5, 6), mode="edge")
    vwinT = jnp.zeros((16, _L), f32).at[:12, :].set(
        jnp.stack([vp[j:j + _L] for j in range(12)], axis=0))

    xpad = jnp.concatenate(
        [jnp.zeros((1, _C), f32), x[0], jnp.zeros((7, _C), f32)], axis=0)

    nblk = _L // _LB
    full = lambda shape: pl.BlockSpec(shape, lambda i: (0, 0))
    out, reg, ent = pl.pallas_call(
        _block_kernel,
        grid=(nblk,),
        in_specs=[
            full((_L + 8, _C)),
            pl.BlockSpec((16, _LB), lambda i: (0, i)),
            full((8, _C)),
            full((_C, _C)),
            full((_C, _K * _C)),
            full((_C, _K * _C)),
            full((_C, 8)),
            full((_C, 8)),
            full((8, 128)),
            full((_C, 8)),
            full((_C, _C)),
            full((1, _C)),
        ],
        out_specs=[
            pl.BlockSpec((_LB, _C), lambda i: (i, 0)),
            pl.BlockSpec((1, 1), lambda i: (0, 0)),
            pl.BlockSpec((1, 1), lambda i: (0, 0)),
        ],
        out_shape=[
            jax.ShapeDtypeStruct((_L, _C), f32),
            jax.ShapeDtypeStruct((1, 1), f32),
            jax.ShapeDtypeStruct((1, 1), f32),
        ],
        interpret=interpret,
    )(xpad, vwinT, dwT, pw, woff2, wmask2, boffT, bmsT, env8, kwtT,
      wout, boutr)

    return out.reshape(1, _L, _C), reg[0, 0], ent[0, 0]


def kernel(x, params):
    return _run(x, params)
